# baseline (speedup 1.0000x reference)
import math

import jax
import jax.numpy as jnp
import numpy as np
from jax import lax
from jax.experimental import pallas as pl
from jax.experimental.pallas import tpu as pltpu
from jax.experimental.pallas import tpu_sc as plsc

D_MODEL = 768
BATCH = 4
SEQ = 8192
N_TOK = BATCH * SEQ
NUM_WORKERS = 32
P_PER_W = SEQ // NUM_WORKERS
CP = 8
ROWS = BATCH * CP
N_CHUNKS = P_PER_W // CP
LANES = 16
SCALE = math.sqrt(float(D_MODEL))


PE_WORDS = D_MODEL // 2


def _pe_table() -> np.ndarray:
    pos = np.arange(SEQ, dtype=np.float32)[:, None]
    div = np.exp(
        np.arange(0, D_MODEL, 2, dtype=np.float32)
        * (-(math.log(10000.0) / D_MODEL))
    )
    angle = (pos * div).astype(np.float32)
    pe = np.empty((SEQ, D_MODEL), dtype=np.float32)
    pe[:, 0::2] = np.sin(angle)
    pe[:, 1::2] = np.cos(angle)
    import ml_dtypes
    u = pe.astype(ml_dtypes.bfloat16).view(np.uint16)
    u = u.reshape(SEQ, D_MODEL // 32, 2, 16)
    w = u[:, :, 0, :].astype(np.uint32) | (
        u[:, :, 1, :].astype(np.uint32) << 16)
    return w.reshape(-1).view(np.int32)


_PE = _pe_table()


def _sc_embed(x_hbm, pe_hbm, lut_hbm, out_hbm,
              idx_v, rows0, rows1, pe0, pe1, out0, out1,
              gsem0, gsem1, psem0, psem1, wsem0, wsem1):
    rows = (rows0, rows1)
    pes = (pe0, pe1)
    outs = (out0, out1)
    gsems = (gsem0, gsem1)
    psems = (psem0, psem1)
    wsems = (wsem0, wsem1)

    wid = lax.axis_index("s") * 2 + lax.axis_index("c")
    p_base = wid * P_PER_W
    pltpu.sync_copy(x_hbm.at[wid], idx_v)

    def start_gather(c, b):
        pltpu.async_copy(lut_hbm.at[idx_v.at[c]], rows[b], gsems[b])
        pltpu.async_copy(
            pe_hbm.at[pl.ds((p_base + c * CP) * PE_WORDS, CP * PE_WORDS)],
            pes[b], psems[b])

    def wait_gather(c, b):
        pltpu.make_async_copy(lut_hbm.at[idx_v.at[c]], rows[b], gsems[b]).wait()
        pltpu.make_async_copy(
            pe_hbm.at[pl.ds((p_base + c * CP) * PE_WORDS, CP * PE_WORDS)],
            pes[b], psems[b]).wait()

    def out_copy(c, b, bb):
        return pltpu.make_async_copy(
            outs[b].at[pl.ds(bb * CP, CP)],
            out_hbm.at[pl.ds(bb * SEQ + p_base + c * CP, CP)],
            wsems[b])

    start_gather(0, 0)
    start_gather(1, 1)

    def step(c, b):
        wait_gather(c, b)

        @pl.when(c >= 2)
        def _():
            for bb in range(BATCH):
                out_copy(c - 2, b, bb).wait()

        def col_body(k2, _):
            sl0 = pl.ds(k2 * 2 * LANES, LANES)
            sl1 = pl.ds((k2 * 2 + 1) * LANES, LANES)
            for r in range(CP):
                pe_pk = pes[b][pl.ds(r * PE_WORDS + k2 * LANES, LANES)]
                pe_a = lax.bitcast_convert_type(
                    lax.shift_left(pe_pk, jnp.int32(16)), jnp.float32)
                pe_b = lax.bitcast_convert_type(
                    lax.bitwise_and(pe_pk, jnp.int32(-65536)), jnp.float32)
                for bb in range(BATCH):
                    rr = bb * CP + r
                    outs[b][rr, sl0] = rows[b][rr, sl0] * SCALE + pe_a
                    outs[b][rr, sl1] = rows[b][rr, sl1] * SCALE + pe_b
            return 0

        lax.fori_loop(0, D_MODEL // (2 * LANES), col_body, 0)

        for bb in range(BATCH):
            out_copy(c, b, bb).start()

        @pl.when(c + 2 < N_CHUNKS)
        def _():
            start_gather(c + 2, b)

    def pair(i, _):
        step(i * 2, 0)
        step(i * 2 + 1, 1)
        return 0

    lax.fori_loop(0, N_CHUNKS // 2, pair, 0)
    for bb in range(BATCH):
        out_copy(N_CHUNKS - 2, 0, bb).wait()
        out_copy(N_CHUNKS - 1, 1, bb).wait()


def kernel(x, lut):
    x_w = (x.astype(jnp.int32)
           .reshape(BATCH, NUM_WORKERS, N_CHUNKS, CP)
           .transpose(1, 2, 0, 3)
           .reshape(NUM_WORKERS, N_CHUNKS, ROWS))
    run = pl.kernel(
        _sc_embed,
        out_type=jax.ShapeDtypeStruct((N_TOK, D_MODEL), jnp.float32),
        mesh=plsc.VectorSubcoreMesh(core_axis_name="c", subcore_axis_name="s"),
        scratch_types=[
            pltpu.VMEM((N_CHUNKS, ROWS), jnp.int32),
            pltpu.VMEM((ROWS, D_MODEL), jnp.float32),
            pltpu.VMEM((ROWS, D_MODEL), jnp.float32),
            pltpu.VMEM((CP * PE_WORDS,), jnp.int32),
            pltpu.VMEM((CP * PE_WORDS,), jnp.int32),
            pltpu.VMEM((ROWS, D_MODEL), jnp.float32),
            pltpu.VMEM((ROWS, D_MODEL), jnp.float32),
            pltpu.SemaphoreType.DMA,
            pltpu.SemaphoreType.DMA,
            pltpu.SemaphoreType.DMA,
            pltpu.SemaphoreType.DMA,
            pltpu.SemaphoreType.DMA,
            pltpu.SemaphoreType.DMA,
        ],
    )
    out = run(x_w, jnp.asarray(_PE), lut)
    return out.reshape(BATCH, SEQ, D_MODEL)


# speedup vs baseline: 2.7050x; 2.7050x over previous
import math

import jax
import jax.numpy as jnp
import numpy as np
from jax import lax
from jax.experimental import pallas as pl
from jax.experimental.pallas import tpu as pltpu
from jax.experimental.pallas import tpu_sc as plsc

D_MODEL = 768
BATCH = 4
SEQ = 8192
N_TOK = BATCH * SEQ
NUM_WORKERS = 32
P_PER_W = SEQ // NUM_WORKERS
CP = 8
ROWS = BATCH * CP
N_CHUNKS = P_PER_W // CP
LANES = 16
SCALE = math.sqrt(float(D_MODEL))


def _pe_table() -> np.ndarray:
    pos = np.arange(SEQ, dtype=np.float32)[:, None]
    div = np.exp(
        np.arange(0, D_MODEL, 2, dtype=np.float32)
        * (-(math.log(10000.0) / D_MODEL))
    )
    angle = (pos * div).astype(np.float32)
    pe = np.empty((SEQ, D_MODEL), dtype=np.float32)
    pe[:, 0::2] = np.sin(angle)
    pe[:, 1::2] = np.cos(angle)
    return pe


_PE = _pe_table()


def _sc_embed(x_hbm, pe_hbm, lut_hbm, out_hbm,
              idx_v, rows0, rows1, pe0, pe1, out0, out1,
              gsem0, gsem1, psem0, psem1, wsem0, wsem1):
    rows = (rows0, rows1)
    pes = (pe0, pe1)
    outs = (out0, out1)
    gsems = (gsem0, gsem1)
    psems = (psem0, psem1)
    wsems = (wsem0, wsem1)

    wid = lax.axis_index("s") * 2 + lax.axis_index("c")
    p_base = wid * P_PER_W
    for bb in range(BATCH):
        pltpu.sync_copy(x_hbm.at[bb, pl.ds(p_base, P_PER_W)], idx_v.at[bb])

    def start_gather(c, b):
        for bb in range(BATCH):
            pltpu.async_copy(
                lut_hbm.at[idx_v.at[bb, pl.ds(c * CP, CP)]],
                rows[b].at[pl.ds(bb * CP, CP)], gsems[b])
        pltpu.async_copy(
            pe_hbm.at[pl.ds(p_base + c * CP, CP)], pes[b], psems[b])

    def wait_gather(c, b):
        for bb in range(BATCH):
            pltpu.make_async_copy(
                lut_hbm.at[idx_v.at[bb, pl.ds(c * CP, CP)]],
                rows[b].at[pl.ds(bb * CP, CP)], gsems[b]).wait()
        pltpu.make_async_copy(
            pe_hbm.at[pl.ds(p_base + c * CP, CP)], pes[b], psems[b]).wait()

    def out_copy(c, b, bb):
        return pltpu.make_async_copy(
            outs[b].at[pl.ds(bb * CP, CP)],
            out_hbm.at[pl.ds(bb * SEQ + p_base + c * CP, CP)],
            wsems[b])

    start_gather(0, 0)
    start_gather(1, 1)

    def step(c, b):
        wait_gather(c, b)

        @pl.when(c >= 2)
        def _():
            for bb in range(BATCH):
                out_copy(c - 2, b, bb).wait()

        def col_body(k, _):
            sl = pl.ds(k * LANES, LANES)
            pev = [pes[b][r, sl] for r in range(CP)]
            for bb in range(BATCH):
                for r in range(CP):
                    rr = bb * CP + r
                    outs[b][rr, sl] = rows[b][rr, sl] * SCALE + pev[r]
            return 0

        lax.fori_loop(0, D_MODEL // LANES, col_body, 0)

        for bb in range(BATCH):
            out_copy(c, b, bb).start()

        @pl.when(c + 2 < N_CHUNKS)
        def _():
            start_gather(c + 2, b)

    def pair(i, _):
        step(i * 2, 0)
        step(i * 2 + 1, 1)
        return 0

    lax.fori_loop(0, N_CHUNKS // 2, pair, 0)
    for bb in range(BATCH):
        out_copy(N_CHUNKS - 2, 0, bb).wait()
        out_copy(N_CHUNKS - 1, 1, bb).wait()


def kernel(x, lut):
    pe = jnp.asarray(_PE)
    run = pl.kernel(
        _sc_embed,
        out_type=jax.ShapeDtypeStruct((N_TOK, D_MODEL), jnp.float32),
        mesh=plsc.VectorSubcoreMesh(core_axis_name="c", subcore_axis_name="s"),
        scratch_types=[
            pltpu.VMEM((BATCH, P_PER_W), jnp.int32),
            pltpu.VMEM((ROWS, D_MODEL), jnp.float32),
            pltpu.VMEM((ROWS, D_MODEL), jnp.float32),
            pltpu.VMEM((CP, D_MODEL), jnp.float32),
            pltpu.VMEM((CP, D_MODEL), jnp.float32),
            pltpu.VMEM((ROWS, D_MODEL), jnp.float32),
            pltpu.VMEM((ROWS, D_MODEL), jnp.float32),
            pltpu.SemaphoreType.DMA,
            pltpu.SemaphoreType.DMA,
            pltpu.SemaphoreType.DMA,
            pltpu.SemaphoreType.DMA,
            pltpu.SemaphoreType.DMA,
            pltpu.SemaphoreType.DMA,
        ],
    )
    out = run(x.astype(jnp.int32), pe, lut)
    return out.reshape(BATCH, SEQ, D_MODEL)
